# Initial kernel scaffold; baseline (speedup 1.0000x reference)
import jax
import jax.numpy as jnp
from jax.experimental import pallas as pl


def kernel(x, pos_table):
    raise NotImplementedError("write your pallas kernel here")


# speedup vs baseline: 1.2770x; 1.2770x over previous
import jax
import jax.numpy as jnp
from jax.experimental import pallas as pl

BLK_S = 512


def _add_kernel(x_ref, pos_ref, o_ref):
    o_ref[...] = x_ref[...] + pos_ref[...]


def kernel(x, pos_table):
    batch, seq_len, embed = x.shape
    grid = (batch, seq_len // BLK_S)
    return pl.pallas_call(
        _add_kernel,
        grid=grid,
        in_specs=[
            pl.BlockSpec((1, BLK_S, embed), lambda b, s: (b, s, 0)),
            pl.BlockSpec((BLK_S, embed), lambda b, s: (s, 0)),
        ],
        out_specs=pl.BlockSpec((1, BLK_S, embed), lambda b, s: (b, s, 0)),
        out_shape=jax.ShapeDtypeStruct((batch, seq_len, embed), x.dtype),
    )(x, pos_table[:seq_len])


# speedup vs baseline: 1.4836x; 1.1618x over previous
import jax
import jax.numpy as jnp
from jax.experimental import pallas as pl

BLK_S = 512


def _add_kernel(x_ref, pos_ref, o_ref):
    o_ref[...] = x_ref[...] + pos_ref[...]


def kernel(x, pos_table):
    batch, seq_len, embed = x.shape
    grid = (seq_len // BLK_S, batch)
    return pl.pallas_call(
        _add_kernel,
        grid=grid,
        in_specs=[
            pl.BlockSpec((1, BLK_S, embed), lambda s, b: (b, s, 0)),
            pl.BlockSpec((BLK_S, embed), lambda s, b: (s, 0)),
        ],
        out_specs=pl.BlockSpec((1, BLK_S, embed), lambda s, b: (b, s, 0)),
        out_shape=jax.ShapeDtypeStruct((batch, seq_len, embed), x.dtype),
    )(x, pos_table[:seq_len])


# speedup vs baseline: 1.6669x; 1.1235x over previous
import jax
import jax.numpy as jnp
from jax.experimental import pallas as pl

BLK_S = 1024


def _add_kernel(x_ref, pos_ref, o_ref):
    o_ref[...] = x_ref[...] + pos_ref[...]


def kernel(x, pos_table):
    batch, seq_len, embed = x.shape
    grid = (seq_len // BLK_S, batch)
    return pl.pallas_call(
        _add_kernel,
        grid=grid,
        in_specs=[
            pl.BlockSpec((1, BLK_S, embed), lambda s, b: (b, s, 0)),
            pl.BlockSpec((BLK_S, embed), lambda s, b: (s, 0)),
        ],
        out_specs=pl.BlockSpec((1, BLK_S, embed), lambda s, b: (b, s, 0)),
        out_shape=jax.ShapeDtypeStruct((batch, seq_len, embed), x.dtype),
    )(x, pos_table[:seq_len])


# speedup vs baseline: 1.7366x; 1.0418x over previous
import jax
import jax.numpy as jnp
from jax.experimental import pallas as pl

BLK_S = 2048


def _add_kernel(x_ref, pos_ref, o_ref):
    o_ref[...] = x_ref[...] + pos_ref[...]


def kernel(x, pos_table):
    batch, seq_len, embed = x.shape
    grid = (seq_len // BLK_S, batch)
    return pl.pallas_call(
        _add_kernel,
        grid=grid,
        in_specs=[
            pl.BlockSpec((1, BLK_S, embed), lambda s, b: (b, s, 0)),
            pl.BlockSpec((BLK_S, embed), lambda s, b: (s, 0)),
        ],
        out_specs=pl.BlockSpec((1, BLK_S, embed), lambda s, b: (b, s, 0)),
        out_shape=jax.ShapeDtypeStruct((batch, seq_len, embed), x.dtype),
    )(x, pos_table[:seq_len])
